# baseline (speedup 1.0000x reference)
import functools

import jax
import jax.numpy as jnp
from jax import lax
from jax.experimental import pallas as pl
from jax.experimental.pallas import tpu as pltpu
from jax.experimental.pallas import tpu_sc as plsc

NC = 2
NS = 16
LANES = 16
CHUNK = 128


def _mm_body(x_ref, w_ref, o_ref):
    o_ref[...] = jnp.dot(x_ref[...], w_ref[...], preferred_element_type=jnp.float32)


def _combine_body(p_ref, b_ref, o_ref):
    o_ref[...] = p_ref[0] + p_ref[1] + b_ref[...]


def _make_sc_agg(n_nodes, chunks_per_tile, d):
    nw = NC * NS
    acc_rows = n_nodes + 8
    rows_per_tile = (n_nodes // NS) & ~7
    rows_tail = n_nodes - NS * rows_per_tile
    mesh = plsc.VectorSubcoreMesh(core_axis_name="c", subcore_axis_name="s")

    @functools.partial(
        pl.kernel,
        out_type=jax.ShapeDtypeStruct((NC, n_nodes, d), jnp.float32),
        mesh=mesh,
        scratch_types=[
            pltpu.VMEM((chunks_per_tile // 2, CHUNK), jnp.int32),
            pltpu.VMEM((chunks_per_tile // 2, CHUNK), jnp.int32),
            pltpu.VMEM((CHUNK, d), jnp.float32),
            pltpu.VMEM((CHUNK, d), jnp.float32),
            pltpu.VMEM_SHARED((acc_rows, d), jnp.float32),
            pltpu.SemaphoreType.DMA,
            pltpu.SemaphoreType.DMA,
        ],
    )
    def sc_agg(y_hbm, src_hbm, dst_hbm, out_hbm,
               src_t, dst_t, buf_a, buf_b, acc_sh, sem_a, sem_b):
        c = lax.axis_index("c")
        s = lax.axis_index("s")
        w = c * NS + s

        def zero_row(i, carry):
            for j in range(d // LANES):
                buf_a[i, pl.ds(j * LANES, LANES)] = jnp.zeros((LANES,), jnp.float32)
            return carry
        lax.fori_loop(0, CHUNK, zero_row, 0)

        row_base = s * rows_per_tile
        n_full = rows_per_tile // CHUNK
        for k in range(n_full):
            pltpu.sync_copy(buf_a, acc_sh.at[pl.ds(row_base + k * CHUNK, CHUNK)])
        tail = rows_per_tile - n_full * CHUNK
        if tail:
            pltpu.sync_copy(buf_a.at[pl.ds(0, tail)],
                            acc_sh.at[pl.ds(row_base + n_full * CHUNK, tail)])
        if rows_tail:
            @pl.when(s == NS - 1)
            def _zero_last_rows():
                pltpu.sync_copy(buf_a.at[pl.ds(0, rows_tail)],
                                acc_sh.at[pl.ds(NS * rows_per_tile, rows_tail)])
        plsc.subcore_barrier()

        hb = chunks_per_tile // 2
        for half in range(2):
            hbase = w * chunks_per_tile + half * hb
            pltpu.sync_copy(src_hbm.at[pl.ds(hbase, hb)], src_t)
            pltpu.sync_copy(dst_hbm.at[pl.ds(hbase, hb)], dst_t)

            pltpu.async_copy(y_hbm.at[src_t.at[0]], buf_a, sem_a)
            pltpu.async_copy(y_hbm.at[src_t.at[1]], buf_b, sem_b)

            def chunk_pair(i, carry):
                pltpu.make_async_copy(y_hbm.at[pl.ds(0, CHUNK)], buf_a, sem_a).wait()
                pltpu.sync_copy(buf_a, acc_sh.at[dst_t.at[2 * i]], add=True)
                pltpu.async_copy(y_hbm.at[src_t.at[2 * i + 2]], buf_a, sem_a)
                pltpu.make_async_copy(y_hbm.at[pl.ds(0, CHUNK)], buf_b, sem_b).wait()
                pltpu.sync_copy(buf_b, acc_sh.at[dst_t.at[2 * i + 1]], add=True)
                pltpu.async_copy(y_hbm.at[src_t.at[2 * i + 3]], buf_b, sem_b)
                return carry
            lax.fori_loop(0, hb // 2 - 1, chunk_pair, 0)

            pltpu.make_async_copy(y_hbm.at[pl.ds(0, CHUNK)], buf_a, sem_a).wait()
            pltpu.sync_copy(buf_a, acc_sh.at[dst_t.at[hb - 2]], add=True)
            pltpu.make_async_copy(y_hbm.at[pl.ds(0, CHUNK)], buf_b, sem_b).wait()
            pltpu.sync_copy(buf_b, acc_sh.at[dst_t.at[hb - 1]], add=True)

        plsc.subcore_barrier()
        pltpu.sync_copy(acc_sh.at[pl.ds(row_base, rows_per_tile)],
                        out_hbm.at[c, pl.ds(row_base, rows_per_tile)])
        if rows_tail:
            @pl.when(s == NS - 1)
            def _copy_last_rows():
                pltpu.sync_copy(acc_sh.at[pl.ds(NS * rows_per_tile, rows_tail)],
                                out_hbm.at[c, pl.ds(NS * rows_per_tile, rows_tail)])

    return sc_agg


def kernel(x, edge_index, W, b):
    n_nodes, d_in = x.shape
    d_out = W.shape[1]
    n_edges = edge_index.shape[1]

    src = edge_index[1].astype(jnp.int32)
    dst = edge_index[0].astype(jnp.int32)

    nw = NC * NS
    cpt = -(-n_edges // (nw * CHUNK))
    cpt = max(2, (cpt + 7) & ~7)
    n_pad = nw * cpt * CHUNK - n_edges
    src_p = jnp.concatenate([src, jnp.zeros((n_pad,), jnp.int32)])
    dst_p = jnp.concatenate([dst, jnp.full((n_pad,), n_nodes, jnp.int32)])
    src2d = src_p.reshape(nw * cpt, CHUNK)
    dst2d = dst_p.reshape(nw * cpt, CHUNK)

    row_blk = 1000
    y = pl.pallas_call(
        _mm_body,
        grid=(n_nodes // row_blk,),
        in_specs=[pl.BlockSpec((row_blk, d_in), lambda i: (i, 0)),
                  pl.BlockSpec((d_in, d_out), lambda i: (0, 0))],
        out_specs=pl.BlockSpec((row_blk, d_out), lambda i: (i, 0)),
        out_shape=jax.ShapeDtypeStruct((n_nodes, d_out), jnp.float32),
    )(x, W)

    partials = _make_sc_agg(n_nodes, cpt, d_out)(y, src2d, dst2d)

    out = pl.pallas_call(
        _combine_body,
        grid=(n_nodes // row_blk,),
        in_specs=[pl.BlockSpec((NC, row_blk, d_out), lambda i: (0, i, 0)),
                  pl.BlockSpec((1, d_out), lambda i: (0, 0))],
        out_specs=pl.BlockSpec((row_blk, d_out), lambda i: (i, 0)),
        out_shape=jax.ShapeDtypeStruct((n_nodes, d_out), jnp.float32),
    )(partials, b.reshape(1, d_out))
    return out
